# baseline (speedup 1.0000x reference)
import functools

import jax
import jax.numpy as jnp
from jax import lax
from jax.experimental import pallas as pl
from jax.experimental.pallas import tpu as pltpu
from jax.experimental.pallas import tpu_sc as plsc

_C = 128
_NBUF = 4


def _make_emb_kernel(n_chunks, chunks_per_w, emb_dim):
    mesh = plsc.VectorSubcoreMesh(core_axis_name="c", subcore_axis_name="s")
    num_cores = mesh.num_cores

    @functools.partial(
        pl.kernel,
        out_type=jax.ShapeDtypeStruct((n_chunks * _C, emb_dim), jnp.float32),
        mesh=mesh,
        scratch_types=[
            pltpu.VMEM((chunks_per_w, _C), jnp.int32),
            [pltpu.VMEM((_C, emb_dim), jnp.float32) for _ in range(_NBUF)],
            [pltpu.SemaphoreType.DMA for _ in range(_NBUF)],
        ],
        compiler_params=pltpu.CompilerParams(use_tc_tiling_on_sc=False),
    )
    def emb(x_hbm, table_hbm, out_hbm, idx_v, rows, sems):
        wid = lax.axis_index("s") * num_cores + lax.axis_index("c")
        chunk0 = wid * chunks_per_w
        pltpu.sync_copy(x_hbm.at[pl.ds(chunk0, chunks_per_w)], idx_v)

        def gather_chunk(i, b):
            return pltpu.async_copy(table_hbm.at[idx_v.at[i]], rows[b], sems[b])

        def drain_chunk(i, b):
            pltpu.make_async_copy(
                table_hbm.at[idx_v.at[i]], rows[b], sems[b]
            ).wait()
            pltpu.sync_copy(
                rows[b], out_hbm.at[pl.ds((chunk0 + i) * _C, _C)]
            )

        for b in range(_NBUF):
            gather_chunk(b, b)

        def body(k, carry):
            for b in range(_NBUF):
                i = k * _NBUF + b
                drain_chunk(i, b)
                gather_chunk(i + _NBUF, b)
            return carry

        n_full = chunks_per_w // _NBUF - 1
        lax.fori_loop(0, n_full, body, 0, unroll=False)

        for b in range(_NBUF):
            drain_chunk(n_full * _NBUF + b, b)

    return emb


def kernel(x, table):
    b0, b1 = x.shape
    vocab, emb_dim = table.shape
    n = b0 * b1
    n_chunks = n // _C
    n_workers = 32
    chunks_per_w = n_chunks // n_workers
    xf = x.reshape(n_chunks, _C).astype(jnp.int32)
    emb = _make_emb_kernel(n_chunks, chunks_per_w, emb_dim)
    out = emb(xf, table)
    return out.reshape(b0, b1, emb_dim)
